# baseline (speedup 1.0000x reference)
import functools

import jax
import jax.numpy as jnp
from jax import lax
from jax.experimental import pallas as pl
from jax.experimental.pallas import tpu as pltpu
from jax.experimental.pallas import tpu_sc as plsc

N_WORKERS = 32
CHUNK = 40
NBUF = 5
BE = 3200
NSEG = 2


def _sc_gather_concat(atom_features, idx0, idx1):
    E = idx0.shape[0]
    D = atom_features.shape[1]
    dt = atom_features.dtype
    per_w = E // N_WORKERS
    n_chunks = per_w // CHUNK
    assert n_chunks % NBUF == 0
    n_outer = n_chunks // NBUF
    mesh = plsc.VectorSubcoreMesh(core_axis_name="c", subcore_axis_name="s")

    def body(atom_hbm, idx0_hbm, idx1_hbm, out_hbm, idx_v, rows_v, *sems):
        sem_i = sems[:2 * NBUF]
        sem_g = sems[2 * NBUF:4 * NBUF]
        sem_s = sems[4 * NBUF:]
        cid = lax.axis_index("c")
        sid = lax.axis_index("s")
        wid = sid * 2 + cid
        base_w = wid * per_w

        def step(k, carry):
            base0 = base_w + k * NBUF * CHUNK
            idx_srcs = []
            for b in range(NBUF):
                for h, idx_hbm in enumerate((idx0_hbm, idx1_hbm)):
                    src = idx_hbm.at[pl.ds(base0 + b * CHUNK, CHUNK)]
                    idx_srcs.append(src)
                    pltpu.async_copy(src, idx_v.at[2 * b + h], sem_i[2 * b + h])
            gathers = []
            for u in range(2 * NBUF):
                pltpu.make_async_copy(idx_srcs[u], idx_v.at[u], sem_i[u]).wait()
                gathers.append(
                    pltpu.async_copy(atom_hbm.at[idx_v.at[u]], rows_v.at[u],
                                     sem_g[u]))
            stores = []
            for b in range(NBUF):
                for h in range(2):
                    u = 2 * b + h
                    gathers[u].wait()
                    stores.append(
                        pltpu.async_copy(
                            rows_v.at[u],
                            out_hbm.at[pl.ds(base0 + b * CHUNK, CHUNK),
                                       pl.ds(h * D, D)],
                            sem_s[u]))
            for st in stores:
                st.wait()
            return carry

        lax.fori_loop(0, n_outer, step, 0)

    k = pl.kernel(
        body,
        out_type=jax.ShapeDtypeStruct((E, 2 * D), dt),
        mesh=mesh,
        scratch_types=(
            [pltpu.VMEM((2 * NBUF, CHUNK), jnp.int32),
             pltpu.VMEM((2 * NBUF, CHUNK, D), dt)]
            + [pltpu.SemaphoreType.DMA] * (6 * NBUF)
        ),
    )
    return k(atom_features, idx0, idx1)


def _tc_mlp_body(x_ref, bd_ref, wab, wc, b1r, w2, b2r, g2w, gb2r,
                 prev_ref, o_ref):
    del prev_ref
    x = x_ref[...].astype(jnp.bfloat16)
    bd = bd_ref[...].astype(jnp.bfloat16)
    t = (jnp.dot(x, wab[...], preferred_element_type=jnp.float32)
         + jnp.dot(bd, wc[...], preferred_element_type=jnp.float32)
         + b1r[...])
    act = t + t * jnp.tanh(t)
    dh = act.shape[1] // 2
    h = act[:, :dh].astype(jnp.bfloat16)
    g = act[:, dh:].astype(jnp.bfloat16)
    t2 = jnp.dot(h, w2[...], preferred_element_type=jnp.float32) + b2r[...]
    h2 = t2 + t2 * jnp.tanh(t2)
    tg = jnp.dot(g, g2w[...], preferred_element_type=jnp.float32) + gb2r[...]
    o_ref[...] = 0.5 * (h2 + h2 * jnp.tanh(tg))


def _tc_mlp_segment(x_seg, bond, prev_out, block_off, weights):
    Wab, Wc, b1g, W2h, b2h, G2h, gb2h = weights
    E = bond.shape[0]
    DX = x_seg.shape[1]
    DE = bond.shape[1]
    DH2 = Wab.shape[1]
    DH = W2h.shape[0]
    DO = W2h.shape[1]
    nb_seg = x_seg.shape[0] // BE

    def full(shape):
        return pl.BlockSpec(shape, lambda i: (0, 0))

    return pl.pallas_call(
        _tc_mlp_body,
        grid=(nb_seg,),
        in_specs=[
            pl.BlockSpec((BE, DX), lambda i: (i, 0)),
            pl.BlockSpec((BE, DE), lambda i: (i + block_off, 0)),
            full((DX, DH2)), full((DE, DH2)), full((1, DH2)),
            full((DH, DO)), full((1, DO)),
            full((DH, DO)), full((1, DO)),
            pl.BlockSpec(memory_space=pl.ANY),
        ],
        out_specs=pl.BlockSpec((BE, DO), lambda i: (i + block_off, 0)),
        out_shape=jax.ShapeDtypeStruct((E, DO), jnp.float32),
        input_output_aliases={9: 0},
    )(x_seg, bond, Wab, Wc, b1g, W2h, b2h, G2h, gb2h, prev_out)


def kernel(atom_features, bond_features, bond_atom_indices,
           W1, b1, W2, b2, G1, gb1, G2, gb2):
    D = atom_features.shape[1]
    E = bond_features.shape[0]
    seg = E // NSEG
    idx0 = bond_atom_indices[:, 0]
    idx1 = bond_atom_indices[:, 1]
    bf = jnp.bfloat16
    weights = (
        (0.5 * jnp.concatenate([W1[:2 * D], G1[:2 * D]], axis=1)).astype(bf),
        (0.5 * jnp.concatenate([W1[2 * D:], G1[2 * D:]], axis=1)).astype(bf),
        0.5 * jnp.concatenate([b1, gb1])[None, :],
        (0.5 * W2).astype(bf), 0.5 * b2[None, :],
        (0.5 * G2).astype(bf), 0.5 * gb2[None, :],
    )
    xs = [_sc_gather_concat(atom_features,
                            idx0[k * seg:(k + 1) * seg],
                            idx1[k * seg:(k + 1) * seg])
          for k in range(NSEG)]
    out = jnp.zeros((E, W2.shape[1]), dtype=jnp.float32)
    for k in range(NSEG):
        out = _tc_mlp_segment(xs[k], bond_features, out,
                              k * (seg // BE), weights)
    return out


# speedup vs baseline: 1.0701x; 1.0701x over previous
import functools

import jax
import jax.numpy as jnp
from jax import lax
from jax.experimental import pallas as pl
from jax.experimental.pallas import tpu as pltpu
from jax.experimental.pallas import tpu_sc as plsc

N_WORKERS = 32
CHUNK = 40
NBUF = 5
BE = 3200
NSEG = 2


def _sc_gather_concat(atom_features, idx0, idx1, prev=None):
    E = idx0.shape[0]
    D = atom_features.shape[1]
    dt = atom_features.dtype
    per_w = E // N_WORKERS
    n_chunks = per_w // CHUNK
    assert n_chunks % NBUF == 0
    n_outer = n_chunks // NBUF
    mesh = plsc.VectorSubcoreMesh(core_axis_name="c", subcore_axis_name="s")

    def body(atom_hbm, idx0_hbm, idx1_hbm, *rest):
        if prev is None:
            out_hbm, idx_v, rows_v, *sems = rest
        else:
            _prev_hbm, out_hbm, idx_v, rows_v, *sems = rest
        sem_i = sems[:2 * NBUF]
        sem_g = sems[2 * NBUF:4 * NBUF]
        sem_s = sems[4 * NBUF:]
        cid = lax.axis_index("c")
        sid = lax.axis_index("s")
        wid = sid * 2 + cid
        base_w = wid * per_w

        def step(k, carry):
            base0 = base_w + k * NBUF * CHUNK
            idx_srcs = []
            for b in range(NBUF):
                for h, idx_hbm in enumerate((idx0_hbm, idx1_hbm)):
                    src = idx_hbm.at[pl.ds(base0 + b * CHUNK, CHUNK)]
                    idx_srcs.append(src)
                    pltpu.async_copy(src, idx_v.at[2 * b + h], sem_i[2 * b + h])
            gathers = []
            for u in range(2 * NBUF):
                pltpu.make_async_copy(idx_srcs[u], idx_v.at[u], sem_i[u]).wait()
                gathers.append(
                    pltpu.async_copy(atom_hbm.at[idx_v.at[u]], rows_v.at[u],
                                     sem_g[u]))
            stores = []
            for b in range(NBUF):
                for h in range(2):
                    u = 2 * b + h
                    gathers[u].wait()
                    stores.append(
                        pltpu.async_copy(
                            rows_v.at[u],
                            out_hbm.at[pl.ds(base0 + b * CHUNK, CHUNK),
                                       pl.ds(h * D, D)],
                            sem_s[u]))
            for st in stores:
                st.wait()
            return carry

        lax.fori_loop(0, n_outer, step, 0)

    k = pl.kernel(
        body,
        out_type=jax.ShapeDtypeStruct((E, 2 * D), dt),
        mesh=mesh,
        scratch_types=(
            [pltpu.VMEM((2 * NBUF, CHUNK), jnp.int32),
             pltpu.VMEM((2 * NBUF, CHUNK, D), dt)]
            + [pltpu.SemaphoreType.DMA] * (6 * NBUF)
        ),
    )
    if prev is None:
        return k(atom_features, idx0, idx1)
    return k(atom_features, idx0, idx1, prev)


def _tc_mlp_body(x_ref, bd_ref, wab, wc, b1r, w2, b2r, g2w, gb2r,
                 *rest):
    o_ref = rest[-1]
    x = x_ref[...].astype(jnp.bfloat16)
    bd = bd_ref[...].astype(jnp.bfloat16)
    t = (jnp.dot(x, wab[...], preferred_element_type=jnp.float32)
         + jnp.dot(bd, wc[...], preferred_element_type=jnp.float32)
         + b1r[...])
    act = t + t * jnp.tanh(t)
    dh = act.shape[1] // 2
    h = act[:, :dh].astype(jnp.bfloat16)
    g = act[:, dh:].astype(jnp.bfloat16)
    t2 = jnp.dot(h, w2[...], preferred_element_type=jnp.float32) + b2r[...]
    h2 = t2 + t2 * jnp.tanh(t2)
    tg = jnp.dot(g, g2w[...], preferred_element_type=jnp.float32) + gb2r[...]
    o_ref[...] = 0.5 * (h2 + h2 * jnp.tanh(tg))


def _tc_mlp_segment(x_seg, bond, prev_out, block_off, weights):
    Wab, Wc, b1g, W2h, b2h, G2h, gb2h = weights
    E = bond.shape[0]
    DX = x_seg.shape[1]
    DE = bond.shape[1]
    DH2 = Wab.shape[1]
    DH = W2h.shape[0]
    DO = W2h.shape[1]
    nb_seg = x_seg.shape[0] // BE

    def full(shape):
        return pl.BlockSpec(shape, lambda i: (0, 0))

    in_specs = [
        pl.BlockSpec((BE, DX), lambda i: (i, 0)),
        pl.BlockSpec((BE, DE), lambda i: (i + block_off, 0)),
        full((DX, DH2)), full((DE, DH2)), full((1, DH2)),
        full((DH, DO)), full((1, DO)),
        full((DH, DO)), full((1, DO)),
    ]
    args = [x_seg, bond, Wab, Wc, b1g, W2h, b2h, G2h, gb2h]
    aliases = {}
    if prev_out is not None:
        in_specs.append(pl.BlockSpec(memory_space=pl.ANY))
        args.append(prev_out)
        aliases = {9: 0}
    return pl.pallas_call(
        _tc_mlp_body,
        grid=(nb_seg,),
        in_specs=in_specs,
        out_specs=pl.BlockSpec((BE, DO), lambda i: (i + block_off, 0)),
        out_shape=jax.ShapeDtypeStruct((E, DO), jnp.float32),
        input_output_aliases=aliases,
    )(*args)


def kernel(atom_features, bond_features, bond_atom_indices,
           W1, b1, W2, b2, G1, gb1, G2, gb2):
    D = atom_features.shape[1]
    E = bond_features.shape[0]
    seg = E // NSEG
    idx0 = bond_atom_indices[:, 0]
    idx1 = bond_atom_indices[:, 1]
    bf = jnp.bfloat16
    weights = (
        (0.5 * jnp.concatenate([W1[:2 * D], G1[:2 * D]], axis=1)).astype(bf),
        (0.5 * jnp.concatenate([W1[2 * D:], G1[2 * D:]], axis=1)).astype(bf),
        0.5 * jnp.concatenate([b1, gb1])[None, :],
        (0.5 * W2).astype(bf), 0.5 * b2[None, :],
        (0.5 * G2).astype(bf), 0.5 * gb2[None, :],
    )
    xs = []
    for k in range(NSEG):
        xs.append(_sc_gather_concat(atom_features,
                                    idx0[k * seg:(k + 1) * seg],
                                    idx1[k * seg:(k + 1) * seg],
                                    prev=xs[-1] if xs else None))
    out = None
    for k in range(NSEG):
        out = _tc_mlp_segment(xs[k], bond_features, out,
                              k * (seg // BE), weights)
    return out
